# Initial kernel scaffold; baseline (speedup 1.0000x reference)
import jax
import jax.numpy as jnp
from jax.experimental import pallas as pl


def kernel(subgraph, feat, norm, W1, b1, W2, b2):
    raise NotImplementedError("write your pallas kernel here")


# speedup vs baseline: 106.4860x; 106.4860x over previous
import functools

import jax
import jax.numpy as jnp
from jax import lax
from jax.experimental import pallas as pl
from jax.experimental.pallas import tpu as pltpu
from jax.experimental.pallas import tpu_sc as plsc

N = 100000
E = 6400000
NPAD = 100096
SLICE = NPAD // 16
VSTEPS = SLICE // 16
NG = E // 128
G = 24
NCHUNK = 65

@functools.lru_cache(maxsize=None)
def _mesh():
    return plsc.VectorSubcoreMesh(core_axis_name="c", subcore_axis_name="s",
                                  num_cores=2, num_subcores=16)


def _edge_pass(sub_ref, table, acc, src_b, dst_b, val_b, sem, w):
    gbase = (w * 195 + jnp.minimum(w, 10)) * 8

    def rowbody(r, _):
        pltpu.async_copy(table.at[src_b.at[r]], val_b.at[r], sem).wait()
        pltpu.sync_copy(val_b.at[r], acc.at[dst_b.at[r]], add=True)
        return 0

    for i in range(NCHUNK):
        g0 = gbase + i * G
        pltpu.sync_copy(sub_ref.at[0, pl.ds(g0, G)], src_b)
        pltpu.sync_copy(sub_ref.at[1, pl.ds(g0, G)], dst_b)
        lax.fori_loop(0, G, rowbody, 0)

    @pl.when(w < 10)
    def _tail():
        g0 = gbase + NCHUNK * G
        pltpu.sync_copy(sub_ref.at[0, pl.ds(g0, 8)], src_b.at[pl.ds(0, 8)])
        pltpu.sync_copy(sub_ref.at[1, pl.ds(g0, 8)], dst_b.at[pl.ds(0, 8)])
        lax.fori_loop(0, 8, rowbody, 0)


def _zero_into(acc, vb, sl):
    def zbody(j, _):
        vb[pl.ds(j * 16, 16)] = jnp.zeros((16,), jnp.float32)
        return 0

    lax.fori_loop(0, VSTEPS, zbody, 0)
    pltpu.sync_copy(vb, acc.at[pl.ds(sl, SLICE)])


def _layer1_body(sub_ref, feat_ref, norm_ref, out_ref,
            table, acc, vb_a, vb_b, src_b, dst_b, val_b, sem):
    c = lax.axis_index("c")
    s = lax.axis_index("s")
    w = c * 16 + s
    sl = s * SLICE
    pltpu.sync_copy(feat_ref.at[pl.ds(sl, SLICE)], vb_a)
    pltpu.sync_copy(norm_ref.at[pl.ds(sl, SLICE)], vb_b)

    def mbody(j, _):
        ix = pl.ds(j * 16, 16)
        vb_a[ix] = vb_a[ix] * vb_b[ix]
        return 0

    lax.fori_loop(0, VSTEPS, mbody, 0)
    pltpu.sync_copy(vb_a, table.at[pl.ds(sl, SLICE)])
    _zero_into(acc, vb_b, sl)
    plsc.subcore_barrier()

    _edge_pass(sub_ref, table, acc, src_b, dst_b, val_b, sem, w)

    plsc.subcore_barrier()
    pltpu.sync_copy(acc.at[pl.ds(sl, SLICE)], vb_a)
    pltpu.sync_copy(vb_a, out_ref.at[pl.ds(c * NPAD + sl, SLICE)])


def _layer2_body(sub_ref, p_ref, norm_ref, w_ref, out_ref,
            table, acc, vb_a, vb_b, vb_c, wb, src_b, dst_b, val_b, sem):
    c = lax.axis_index("c")
    s = lax.axis_index("s")
    w = c * 16 + s
    sl = s * SLICE
    pltpu.sync_copy(p_ref.at[pl.ds(sl, SLICE)], vb_a)
    pltpu.sync_copy(p_ref.at[pl.ds(NPAD + sl, SLICE)], vb_b)
    pltpu.sync_copy(norm_ref.at[pl.ds(sl, SLICE)], vb_c)
    pltpu.sync_copy(w_ref, wb)
    wv = [wb[pl.ds(k * 16, 16)] for k in range(9)]

    def sbody(j, _):
        ix = pl.ds(j * 16, 16)
        nv = vb_c[ix]
        t = (vb_a[ix] + vb_b[ix]) * nv
        sv = jnp.zeros((16,), jnp.float32)
        for k in range(3):
            sv = sv + jnp.maximum(t * wv[k] + wv[3 + k], 0.0) * wv[6 + k]
        vb_a[ix] = sv * nv
        return 0

    lax.fori_loop(0, VSTEPS, sbody, 0)
    pltpu.sync_copy(vb_a, table.at[pl.ds(sl, SLICE)])
    _zero_into(acc, vb_b, sl)
    plsc.subcore_barrier()

    _edge_pass(sub_ref, table, acc, src_b, dst_b, val_b, sem, w)

    plsc.subcore_barrier()
    pltpu.sync_copy(acc.at[pl.ds(sl, SLICE)], vb_a)
    pltpu.sync_copy(vb_a, out_ref.at[pl.ds(c * NPAD + sl, SLICE)])


def _fin_body(p_ref, n_ref, b_ref, o_ref):
    o_ref[...] = jnp.maximum((p_ref[0] + p_ref[1]) * n_ref[...] + b_ref[...], 0.0)


_finalize = pl.pallas_call(
    _fin_body,
    out_shape=jax.ShapeDtypeStruct((NPAD // 128, 128), jnp.float32),
)


@functools.lru_cache(maxsize=None)
def _build_layers():
    common = [
        pltpu.MemorySpace.VMEM_SHARED((NPAD,), jnp.float32),
        pltpu.MemorySpace.VMEM_SHARED((NPAD,), jnp.float32),
    ]
    bufs = [
        pltpu.MemorySpace.VMEM((G, 128), jnp.int32),
        pltpu.MemorySpace.VMEM((G, 128), jnp.int32),
        pltpu.MemorySpace.VMEM((G, 128), jnp.float32),
        pltpu.SemaphoreType.DMA,
    ]
    l1 = pl.kernel(
        _layer1_body,
        out_type=jax.ShapeDtypeStruct((2 * NPAD,), jnp.float32),
        mesh=_mesh(),
        scratch_types=common + [
            pltpu.MemorySpace.VMEM((SLICE,), jnp.float32),
            pltpu.MemorySpace.VMEM((SLICE,), jnp.float32),
        ] + bufs,
    )
    l2 = pl.kernel(
        _layer2_body,
        out_type=jax.ShapeDtypeStruct((2 * NPAD,), jnp.float32),
        mesh=_mesh(),
        scratch_types=common + [
            pltpu.MemorySpace.VMEM((SLICE,), jnp.float32),
            pltpu.MemorySpace.VMEM((SLICE,), jnp.float32),
            pltpu.MemorySpace.VMEM((SLICE,), jnp.float32),
            pltpu.MemorySpace.VMEM((144,), jnp.float32),
        ] + bufs,
    )
    return l1, l2


def kernel(subgraph, feat, norm, W1, b1, W2, b2):
    _l1, _l2 = _build_layers()
    sub3 = subgraph.reshape(2, NG, 128).astype(jnp.int32)
    featp = jnp.pad(feat.reshape(N), (0, NPAD - N))
    normp = jnp.pad(norm.reshape(N), (0, NPAD - N))
    p1 = _l1(sub3, featp, normp)
    wtab = jnp.concatenate([
        jnp.broadcast_to(W1.reshape(3, 1), (3, 16)).reshape(-1),
        jnp.broadcast_to(b1.reshape(3, 1), (3, 16)).reshape(-1),
        jnp.broadcast_to(W2.reshape(3, 1), (3, 16)).reshape(-1),
    ])
    p2 = _l2(sub3, p1, normp, wtab)
    b2t = jnp.broadcast_to(b2.reshape(1, 1), (1, 1))
    out = _finalize(p2.reshape(2, NPAD // 128, 128),
                    normp.reshape(NPAD // 128, 128), b2t)
    return out.reshape(NPAD)[:N].reshape(N, 1)


# speedup vs baseline: 293.2749x; 2.7541x over previous
import functools

import jax
import jax.numpy as jnp
from jax import lax
from jax.experimental import pallas as pl
from jax.experimental.pallas import tpu as pltpu
from jax.experimental.pallas import tpu_sc as plsc

N = 100000
E = 6400000
NPAD = 100096
SLICE = NPAD // 16
VSTEPS = SLICE // 16
NG = E // 128
G = 40
NCHUNK = 39

@functools.lru_cache(maxsize=None)
def _mesh():
    return plsc.VectorSubcoreMesh(core_axis_name="c", subcore_axis_name="s",
                                  num_cores=2, num_subcores=16)


def _edge_pass(sub_ref, table, acc, src_b, dst_b, val_b,
               sem_i0, sem_i1, sem_g, sem_s, w):
    gbase = (w * 195 + jnp.minimum(w, 10)) * 8

    def idx_copies(i, p):
        g0 = gbase + i * G
        sem = sem_i0 if p == 0 else sem_i1
        return (pltpu.make_async_copy(sub_ref.at[0, pl.ds(g0, G)], src_b.at[p], sem),
                pltpu.make_async_copy(sub_ref.at[1, pl.ds(g0, G)], dst_b.at[p], sem))

    def gather_copy(p, r):
        return pltpu.make_async_copy(table.at[src_b.at[p, r]], val_b.at[p, r],
                                     sem_g)

    def scatter_copy(p, r):
        return pltpu.make_async_copy(val_b.at[p, r], acc.at[dst_b.at[p, r]],
                                     sem_s)

    def fire_gathers(p):
        lax.fori_loop(0, G, lambda r, _: (gather_copy(p, r).start(), 0)[1], 0)

    def drain_gathers(p):
        lax.fori_loop(0, G, lambda r, _: (gather_copy(p, r).wait(), 0)[1], 0)

    def fire_scatters(p):
        def b(r, _):
            pltpu.async_copy(val_b.at[p, r], acc.at[dst_b.at[p, r]], sem_s,
                             add=True)
            return 0
        lax.fori_loop(0, G, b, 0)

    def drain_scatters(p):
        lax.fori_loop(0, G, lambda r, _: (scatter_copy(p, r).wait(), 0)[1], 0)

    for cpy in idx_copies(0, 0):
        cpy.start()
    for i in range(NCHUNK):
        p = i % 2
        for cpy in idx_copies(i, p):
            cpy.wait()
        fire_gathers(p)
        if i >= 1:
            drain_scatters(1 - p)
        if i + 1 < NCHUNK:
            for cpy in idx_copies(i + 1, 1 - p):
                cpy.start()
        drain_gathers(p)
        fire_scatters(p)
    drain_scatters((NCHUNK - 1) % 2)

    @pl.when(w < 10)
    def _tail():
        g0 = gbase + NCHUNK * G
        pltpu.sync_copy(sub_ref.at[0, pl.ds(g0, 8)], src_b.at[0, pl.ds(0, 8)])
        pltpu.sync_copy(sub_ref.at[1, pl.ds(g0, 8)], dst_b.at[0, pl.ds(0, 8)])

        def rowbody(r, _):
            gather_copy(0, r).start()
            gather_copy(0, r).wait()
            pltpu.sync_copy(val_b.at[0, r], acc.at[dst_b.at[0, r]], add=True)
            return 0

        lax.fori_loop(0, 8, rowbody, 0)


def _zero_into(acc, vb, sl):
    def zbody(j, _):
        vb[pl.ds(j * 16, 16)] = jnp.zeros((16,), jnp.float32)
        return 0

    lax.fori_loop(0, VSTEPS, zbody, 0)
    pltpu.sync_copy(vb, acc.at[pl.ds(sl, SLICE)])


def _layer1_body(sub_ref, feat_ref, norm_ref, out_ref,
                 table, acc, vb_a, vb_b, src_b, dst_b, val_b,
                 sem_i0, sem_i1, sem_g, sem_s):
    c = lax.axis_index("c")
    s = lax.axis_index("s")
    w = c * 16 + s
    sl = s * SLICE
    pltpu.sync_copy(feat_ref.at[pl.ds(sl, SLICE)], vb_a)
    pltpu.sync_copy(norm_ref.at[pl.ds(sl, SLICE)], vb_b)

    def mbody(j, _):
        ix = pl.ds(j * 16, 16)
        vb_a[ix] = vb_a[ix] * vb_b[ix]
        return 0

    lax.fori_loop(0, VSTEPS, mbody, 0)
    pltpu.sync_copy(vb_a, table.at[pl.ds(sl, SLICE)])
    _zero_into(acc, vb_b, sl)
    plsc.subcore_barrier()

    _edge_pass(sub_ref, table, acc, src_b, dst_b, val_b,
               sem_i0, sem_i1, sem_g, sem_s, w)

    plsc.subcore_barrier()
    pltpu.sync_copy(acc.at[pl.ds(sl, SLICE)], vb_a)
    pltpu.sync_copy(vb_a, out_ref.at[pl.ds(c * NPAD + sl, SLICE)])


def _layer2_body(sub_ref, p_ref, norm_ref, w_ref, out_ref,
                 table, acc, vb_a, vb_b, vb_c, wb, src_b, dst_b, val_b,
                 sem_i0, sem_i1, sem_g, sem_s):
    c = lax.axis_index("c")
    s = lax.axis_index("s")
    w = c * 16 + s
    sl = s * SLICE
    pltpu.sync_copy(p_ref.at[pl.ds(sl, SLICE)], vb_a)
    pltpu.sync_copy(p_ref.at[pl.ds(NPAD + sl, SLICE)], vb_b)
    pltpu.sync_copy(norm_ref.at[pl.ds(sl, SLICE)], vb_c)
    pltpu.sync_copy(w_ref, wb)
    wv = [wb[pl.ds(k * 16, 16)] for k in range(9)]

    def sbody(j, _):
        ix = pl.ds(j * 16, 16)
        nv = vb_c[ix]
        t = (vb_a[ix] + vb_b[ix]) * nv
        sv = jnp.zeros((16,), jnp.float32)
        for k in range(3):
            sv = sv + jnp.maximum(t * wv[k] + wv[3 + k], 0.0) * wv[6 + k]
        vb_a[ix] = sv * nv
        return 0

    lax.fori_loop(0, VSTEPS, sbody, 0)
    pltpu.sync_copy(vb_a, table.at[pl.ds(sl, SLICE)])
    _zero_into(acc, vb_b, sl)
    plsc.subcore_barrier()

    _edge_pass(sub_ref, table, acc, src_b, dst_b, val_b,
               sem_i0, sem_i1, sem_g, sem_s, w)

    plsc.subcore_barrier()
    pltpu.sync_copy(acc.at[pl.ds(sl, SLICE)], vb_a)
    pltpu.sync_copy(vb_a, out_ref.at[pl.ds(c * NPAD + sl, SLICE)])


def _fin_body(p_ref, n_ref, b_ref, o_ref):
    o_ref[...] = jnp.maximum((p_ref[0] + p_ref[1]) * n_ref[...] + b_ref[...], 0.0)


_finalize = pl.pallas_call(
    _fin_body,
    out_shape=jax.ShapeDtypeStruct((NPAD // 128, 128), jnp.float32),
)


@functools.lru_cache(maxsize=None)
def _build_layers():
    common = [
        pltpu.MemorySpace.VMEM_SHARED((NPAD,), jnp.float32),
        pltpu.MemorySpace.VMEM_SHARED((NPAD,), jnp.float32),
    ]
    bufs = [
        pltpu.MemorySpace.VMEM((2, G, 128), jnp.int32),
        pltpu.MemorySpace.VMEM((2, G, 128), jnp.int32),
        pltpu.MemorySpace.VMEM((2, G, 128), jnp.float32),
        pltpu.SemaphoreType.DMA,
        pltpu.SemaphoreType.DMA,
        pltpu.SemaphoreType.DMA,
        pltpu.SemaphoreType.DMA,
    ]
    l1 = pl.kernel(
        _layer1_body,
        out_type=jax.ShapeDtypeStruct((2 * NPAD,), jnp.float32),
        mesh=_mesh(),
        scratch_types=common + [
            pltpu.MemorySpace.VMEM((SLICE,), jnp.float32),
            pltpu.MemorySpace.VMEM((SLICE,), jnp.float32),
        ] + bufs,
    )
    l2 = pl.kernel(
        _layer2_body,
        out_type=jax.ShapeDtypeStruct((2 * NPAD,), jnp.float32),
        mesh=_mesh(),
        scratch_types=common + [
            pltpu.MemorySpace.VMEM((SLICE,), jnp.float32),
            pltpu.MemorySpace.VMEM((SLICE,), jnp.float32),
            pltpu.MemorySpace.VMEM((SLICE,), jnp.float32),
            pltpu.MemorySpace.VMEM((144,), jnp.float32),
        ] + bufs,
    )
    return l1, l2


def kernel(subgraph, feat, norm, W1, b1, W2, b2):
    _l1, _l2 = _build_layers()
    sub3 = subgraph.reshape(2, NG, 128).astype(jnp.int32)
    featp = jnp.pad(feat.reshape(N), (0, NPAD - N))
    normp = jnp.pad(norm.reshape(N), (0, NPAD - N))
    p1 = _l1(sub3, featp, normp)
    wtab = jnp.concatenate([
        jnp.broadcast_to(W1.reshape(3, 1), (3, 16)).reshape(-1),
        jnp.broadcast_to(b1.reshape(3, 1), (3, 16)).reshape(-1),
        jnp.broadcast_to(W2.reshape(3, 1), (3, 16)).reshape(-1),
    ])
    p2 = _l2(sub3, p1, normp, wtab)
    b2t = jnp.broadcast_to(b2.reshape(1, 1), (1, 1))
    out = _finalize(p2.reshape(2, NPAD // 128, 128),
                    normp.reshape(NPAD // 128, 128), b2t)
    return out.reshape(NPAD)[:N].reshape(N, 1)


# speedup vs baseline: 305.0033x; 1.0400x over previous
import functools

import jax
import jax.numpy as jnp
from jax import lax
from jax.experimental import pallas as pl
from jax.experimental.pallas import tpu as pltpu
from jax.experimental.pallas import tpu_sc as plsc

N = 100000
E = 6400000
NPAD = 100096
SLICE = NPAD // 16
VSTEPS = SLICE // 16
NG = E // 128
G = 40
NCHUNK = 39

@functools.lru_cache(maxsize=None)
def _mesh():
    return plsc.VectorSubcoreMesh(core_axis_name="c", subcore_axis_name="s",
                                  num_cores=2, num_subcores=16)


CH = G * 128


def _edge_pass(sub_ref, table, acc, src_b, dst_b, val_b,
               sem_i0, sem_i1, sem_g, sem_s, w):
    gbase = (w * 195 + jnp.minimum(w, 10)) * 8

    def idx_copies(i, p):
        e0 = (gbase + i * G) * 128
        sem = sem_i0 if p == 0 else sem_i1
        return (pltpu.make_async_copy(sub_ref.at[0, pl.ds(e0, CH)],
                                      src_b.at[p], sem),
                pltpu.make_async_copy(sub_ref.at[1, pl.ds(e0, CH)],
                                      dst_b.at[p], sem))

    def gather_copy(p):
        return pltpu.make_async_copy(table.at[src_b.at[p]], val_b.at[p],
                                     sem_g)

    def scatter_copy(p):
        return pltpu.make_async_copy(val_b.at[p], acc.at[dst_b.at[p]],
                                     sem_s)

    for cpy in idx_copies(0, 0):
        cpy.start()
    for i in range(NCHUNK):
        p = i % 2
        for cpy in idx_copies(i, p):
            cpy.wait()
        gather_copy(p).start()
        if i >= 1:
            scatter_copy(1 - p).wait()
        if i + 1 < NCHUNK:
            for cpy in idx_copies(i + 1, 1 - p):
                cpy.start()
        gather_copy(p).wait()
        pltpu.async_copy(val_b.at[p], acc.at[dst_b.at[p]], sem_s, add=True)
    scatter_copy((NCHUNK - 1) % 2).wait()

    @pl.when(w < 10)
    def _tail():
        e0 = (gbase + NCHUNK * G) * 128
        pltpu.sync_copy(sub_ref.at[0, pl.ds(e0, 1024)],
                        src_b.at[0, pl.ds(0, 1024)])
        pltpu.sync_copy(sub_ref.at[1, pl.ds(e0, 1024)],
                        dst_b.at[0, pl.ds(0, 1024)])
        pltpu.async_copy(table.at[src_b.at[0, pl.ds(0, 1024)]],
                         val_b.at[0, pl.ds(0, 1024)], sem_g).wait()
        pltpu.sync_copy(val_b.at[0, pl.ds(0, 1024)],
                        acc.at[dst_b.at[0, pl.ds(0, 1024)]], add=True)


def _zero_into(acc, vb, sl):
    def zbody(j, _):
        vb[pl.ds(j * 16, 16)] = jnp.zeros((16,), jnp.float32)
        return 0

    lax.fori_loop(0, VSTEPS, zbody, 0)
    pltpu.sync_copy(vb, acc.at[pl.ds(sl, SLICE)])


def _layer1_body(sub_ref, feat_ref, norm_ref, out_ref,
                 table, acc, vb_a, vb_b, src_b, dst_b, val_b,
                 sem_i0, sem_i1, sem_g, sem_s):
    c = lax.axis_index("c")
    s = lax.axis_index("s")
    w = c * 16 + s
    sl = s * SLICE
    pltpu.sync_copy(feat_ref.at[pl.ds(sl, SLICE)], vb_a)
    pltpu.sync_copy(norm_ref.at[pl.ds(sl, SLICE)], vb_b)

    def mbody(j, _):
        ix = pl.ds(j * 16, 16)
        vb_a[ix] = vb_a[ix] * vb_b[ix]
        return 0

    lax.fori_loop(0, VSTEPS, mbody, 0)
    pltpu.sync_copy(vb_a, table.at[pl.ds(sl, SLICE)])
    _zero_into(acc, vb_b, sl)
    plsc.subcore_barrier()

    _edge_pass(sub_ref, table, acc, src_b, dst_b, val_b,
               sem_i0, sem_i1, sem_g, sem_s, w)

    plsc.subcore_barrier()
    pltpu.sync_copy(acc.at[pl.ds(sl, SLICE)], vb_a)
    pltpu.sync_copy(vb_a, out_ref.at[pl.ds(c * NPAD + sl, SLICE)])


def _layer2_body(sub_ref, p_ref, norm_ref, w_ref, out_ref,
                 table, acc, vb_a, vb_b, vb_c, wb, src_b, dst_b, val_b,
                 sem_i0, sem_i1, sem_g, sem_s):
    c = lax.axis_index("c")
    s = lax.axis_index("s")
    w = c * 16 + s
    sl = s * SLICE
    pltpu.sync_copy(p_ref.at[pl.ds(sl, SLICE)], vb_a)
    pltpu.sync_copy(p_ref.at[pl.ds(NPAD + sl, SLICE)], vb_b)
    pltpu.sync_copy(norm_ref.at[pl.ds(sl, SLICE)], vb_c)
    pltpu.sync_copy(w_ref, wb)
    wv = [wb[pl.ds(k * 16, 16)] for k in range(9)]

    def sbody(j, _):
        ix = pl.ds(j * 16, 16)
        nv = vb_c[ix]
        t = (vb_a[ix] + vb_b[ix]) * nv
        sv = jnp.zeros((16,), jnp.float32)
        for k in range(3):
            sv = sv + jnp.maximum(t * wv[k] + wv[3 + k], 0.0) * wv[6 + k]
        vb_a[ix] = sv * nv
        return 0

    lax.fori_loop(0, VSTEPS, sbody, 0)
    pltpu.sync_copy(vb_a, table.at[pl.ds(sl, SLICE)])
    _zero_into(acc, vb_b, sl)
    plsc.subcore_barrier()

    _edge_pass(sub_ref, table, acc, src_b, dst_b, val_b,
               sem_i0, sem_i1, sem_g, sem_s, w)

    plsc.subcore_barrier()
    pltpu.sync_copy(acc.at[pl.ds(sl, SLICE)], vb_a)
    pltpu.sync_copy(vb_a, out_ref.at[pl.ds(c * NPAD + sl, SLICE)])


def _fin_body(p_ref, n_ref, b_ref, o_ref):
    o_ref[...] = jnp.maximum((p_ref[0] + p_ref[1]) * n_ref[...] + b_ref[...], 0.0)


_finalize = pl.pallas_call(
    _fin_body,
    out_shape=jax.ShapeDtypeStruct((NPAD // 128, 128), jnp.float32),
)


@functools.lru_cache(maxsize=None)
def _build_layers():
    common = [
        pltpu.MemorySpace.VMEM_SHARED((NPAD,), jnp.float32),
        pltpu.MemorySpace.VMEM_SHARED((NPAD,), jnp.float32),
    ]
    bufs = [
        pltpu.MemorySpace.VMEM((2, CH), jnp.int32),
        pltpu.MemorySpace.VMEM((2, CH), jnp.int32),
        pltpu.MemorySpace.VMEM((2, CH), jnp.float32),
        pltpu.SemaphoreType.DMA,
        pltpu.SemaphoreType.DMA,
        pltpu.SemaphoreType.DMA,
        pltpu.SemaphoreType.DMA,
    ]
    cp = pltpu.CompilerParams(use_tc_tiling_on_sc=False)
    l1 = pl.kernel(
        _layer1_body,
        compiler_params=cp,
        out_type=jax.ShapeDtypeStruct((2 * NPAD,), jnp.float32),
        mesh=_mesh(),
        scratch_types=common + [
            pltpu.MemorySpace.VMEM((SLICE,), jnp.float32),
            pltpu.MemorySpace.VMEM((SLICE,), jnp.float32),
        ] + bufs,
    )
    l2 = pl.kernel(
        _layer2_body,
        compiler_params=cp,
        out_type=jax.ShapeDtypeStruct((2 * NPAD,), jnp.float32),
        mesh=_mesh(),
        scratch_types=common + [
            pltpu.MemorySpace.VMEM((SLICE,), jnp.float32),
            pltpu.MemorySpace.VMEM((SLICE,), jnp.float32),
            pltpu.MemorySpace.VMEM((SLICE,), jnp.float32),
            pltpu.MemorySpace.VMEM((144,), jnp.float32),
        ] + bufs,
    )
    return l1, l2


def kernel(subgraph, feat, norm, W1, b1, W2, b2):
    _l1, _l2 = _build_layers()
    featp = jnp.pad(feat.reshape(N), (0, NPAD - N))
    normp = jnp.pad(norm.reshape(N), (0, NPAD - N))
    p1 = _l1(subgraph, featp, normp)
    wtab = jnp.concatenate([
        jnp.broadcast_to(W1.reshape(3, 1), (3, 16)).reshape(-1),
        jnp.broadcast_to(b1.reshape(3, 1), (3, 16)).reshape(-1),
        jnp.broadcast_to(W2.reshape(3, 1), (3, 16)).reshape(-1),
    ])
    p2 = _l2(subgraph, p1, normp, wtab)
    b2t = jnp.broadcast_to(b2.reshape(1, 1), (1, 1))
    out = _finalize(p2.reshape(2, NPAD // 128, 128),
                    normp.reshape(NPAD // 128, 128), b2t)
    return out.reshape(NPAD)[:N].reshape(N, 1)
